# baseline (speedup 1.0000x reference)
import functools
import math

import jax
import jax.numpy as jnp
from jax import lax
from jax.experimental import pallas as pl
from jax.experimental.pallas import tpu as pltpu
from jax.experimental.pallas import tpu_sc as plsc

_EPS = 1e-5
_L = 16
_IDX_CHUNK = 128


_GATHER_DNUMS = lax.GatherDimensionNumbers(
    offset_dims=(), collapsed_slice_dims=(0,), start_index_map=(0,))


def _shuffle16(x, idx):
  return lax.gather(
      x,
      idx[:, None],
      _GATHER_DNUMS,
      slice_sizes=(1,),
      mode=lax.GatherScatterMode.PROMISE_IN_BOUNDS)


def _rsqrt16(x):
  i = lax.bitcast_convert_type(x, jnp.int32)
  i = jnp.full((_L,), 0x5F3759DF, jnp.int32) - lax.shift_right_arithmetic(
      i, jnp.full((_L,), 1, jnp.int32))
  y = lax.bitcast_convert_type(i, jnp.float32)
  half_x = x * 0.5
  for _ in range(3):
    y = y * (1.5 - half_x * y * y)
  return y


@functools.partial(jax.jit, static_argnums=(0, 1, 2, 3))
def _embed_ln_call(n_tokens, seq_len, d, n_workers, ids2d, table, pe, gamma,
                   beta):
  rows_per_w = n_tokens // n_workers
  n_chunks = rows_per_w // _IDX_CHUNK
  n_vecs = d // _L
  scale = math.sqrt(float(d))
  mesh = plsc.VectorSubcoreMesh(core_axis_name="c", subcore_axis_name="s")

  @functools.partial(
      pl.kernel,
      mesh=mesh,
      out_type=jax.ShapeDtypeStruct((n_tokens, d), jnp.float32),
      scratch_types=[
          pltpu.VMEM((n_chunks, _IDX_CHUNK), jnp.int32),
          pltpu.VMEM((rows_per_w, d), jnp.float32),
          pltpu.VMEM((rows_per_w, d), jnp.float32),
          pltpu.VMEM((d,), jnp.float32),
          pltpu.VMEM((d,), jnp.float32),
          pltpu.SemaphoreType.DMA,
      ],
  )
  def body(ids_hbm, table_hbm, pe_hbm, gamma_hbm, beta_hbm, out_hbm, idx_v,
           rows_v, pe_v, g_v, b_v, sem):
    nc = lax.axis_size("c")
    wid = lax.axis_index("s") * nc + lax.axis_index("c")
    base = wid * rows_per_w
    pos0 = lax.rem(base, seq_len)

    pltpu.sync_copy(ids_hbm.at[pl.ds(wid * n_chunks, n_chunks)], idx_v)
    copies = []
    for c in range(n_chunks):
      copies.append(
          pltpu.async_copy(table_hbm.at[idx_v.at[c]],
                           rows_v.at[pl.ds(c * _IDX_CHUNK, _IDX_CHUNK)], sem))
    pltpu.sync_copy(pe_hbm.at[pl.ds(pos0, rows_per_w)], pe_v)
    pltpu.sync_copy(gamma_hbm, g_v)
    pltpu.sync_copy(beta_hbm, b_v)
    for cp in copies:
      cp.wait()

    gs = [g_v[pl.ds(j * _L, _L)] for j in range(n_vecs)]
    bs = [b_v[pl.ds(j * _L, _L)] for j in range(n_vecs)]
    inv_d = 1.0 / float(d)
    lane = lax.broadcasted_iota(jnp.int32, (_L,), 0)
    bfly_idx = [lax.bitwise_xor(lane, jnp.full((_L,), k, jnp.int32))
                for k in (1, 2, 4, 8)]

    def lane_sum(x):
      for idx in bfly_idx:
        x = x + _shuffle16(x, idx)
      return x

    def ln_row(r, carry):
      vs = []
      acc = jnp.zeros((_L,), jnp.float32)
      acc2 = jnp.zeros((_L,), jnp.float32)
      for j in range(n_vecs):
        v = rows_v[r, pl.ds(j * _L, _L)] * scale + pe_v[r, pl.ds(j * _L, _L)]
        vs.append(v)
        acc = acc + v
        acc2 = acc2 + v * v
      mean_v = lane_sum(acc) * inv_d
      ex2_v = lane_sum(acc2) * inv_d
      var_v = ex2_v - mean_v * mean_v
      rstd_v = _rsqrt16(var_v + _EPS)
      for j in range(n_vecs):
        a = gs[j] * rstd_v
        c0 = bs[j] - mean_v * a
        rows_v[r, pl.ds(j * _L, _L)] = vs[j] * a + c0
      return carry

    lax.fori_loop(0, rows_per_w, ln_row, 0)
    pltpu.sync_copy(rows_v, out_hbm.at[pl.ds(base, rows_per_w)])

  return body(ids2d, table, pe, gamma, beta)


def kernel(input_ids, token_table, pe, ln_gamma, ln_beta):
  b, s = input_ids.shape
  v, d = token_table.shape
  n_tokens = b * s
  info = plsc.get_sparse_core_info()
  n_workers = info.num_cores * info.num_subcores
  ids2d = input_ids.reshape(n_tokens // _IDX_CHUNK, _IDX_CHUNK).astype(
      jnp.int32)
  out = _embed_ln_call(n_tokens, s, d, n_workers, ids2d, token_table, pe,
                       ln_gamma, ln_beta)
  return out.reshape(b, s, d)


# speedup vs baseline: 1.1791x; 1.1791x over previous
import functools
import math

import jax
import jax.numpy as jnp
from jax import lax
from jax.experimental import pallas as pl
from jax.experimental.pallas import tpu as pltpu
from jax.experimental.pallas import tpu_sc as plsc

_EPS = 1e-5
_L = 16
_IDX_CHUNK = 128


_GATHER_DNUMS = lax.GatherDimensionNumbers(
    offset_dims=(), collapsed_slice_dims=(0,), start_index_map=(0,))


def _shuffle16(x, idx):
  return lax.gather(
      x,
      idx[:, None],
      _GATHER_DNUMS,
      slice_sizes=(1,),
      mode=lax.GatherScatterMode.PROMISE_IN_BOUNDS)


def _rsqrt16(x):
  i = lax.bitcast_convert_type(x, jnp.int32)
  i = jnp.full((_L,), 0x5F3759DF, jnp.int32) - lax.shift_right_arithmetic(
      i, jnp.full((_L,), 1, jnp.int32))
  y = lax.bitcast_convert_type(i, jnp.float32)
  half_x = x * 0.5
  for _ in range(2):
    y = y * (1.5 - half_x * y * y)
  return y


@functools.partial(jax.jit, static_argnums=(0, 1, 2, 3))
def _embed_ln_call(n_tokens, seq_len, d, n_workers, ids2d, table, pe, gamma,
                   beta):
  rows_per_w = n_tokens // n_workers
  n_chunks = rows_per_w // _IDX_CHUNK
  n_vecs = d // _L
  scale = math.sqrt(float(d))
  mesh = plsc.VectorSubcoreMesh(core_axis_name="c", subcore_axis_name="s")

  @functools.partial(
      pl.kernel,
      mesh=mesh,
      out_type=jax.ShapeDtypeStruct((n_tokens, d), jnp.float32),
      scratch_types=[
          pltpu.VMEM((n_chunks, _IDX_CHUNK), jnp.int32),
          pltpu.VMEM((rows_per_w, d), jnp.float32),
          pltpu.VMEM((rows_per_w, d), jnp.float32),
          pltpu.VMEM((d,), jnp.float32),
          pltpu.VMEM((d,), jnp.float32),
          pltpu.SemaphoreType.DMA,
      ],
  )
  def body(ids_hbm, table_hbm, pe_hbm, gamma_hbm, beta_hbm, out_hbm, idx_v,
           rows_v, pe_v, g_v, b_v, sem):
    nc = lax.axis_size("c")
    wid = lax.axis_index("s") * nc + lax.axis_index("c")
    base = wid * rows_per_w
    pos0 = lax.rem(base, seq_len)

    pltpu.sync_copy(ids_hbm.at[pl.ds(wid * n_chunks, n_chunks)], idx_v)
    copies = []
    for c in range(n_chunks):
      copies.append(
          pltpu.async_copy(table_hbm.at[idx_v.at[c]],
                           rows_v.at[pl.ds(c * _IDX_CHUNK, _IDX_CHUNK)], sem))
    pltpu.sync_copy(pe_hbm.at[pl.ds(pos0, rows_per_w)], pe_v)
    pltpu.sync_copy(gamma_hbm, g_v)
    pltpu.sync_copy(beta_hbm, b_v)
    for cp in copies:
      cp.wait()

    gs = [g_v[pl.ds(j * _L, _L)] for j in range(n_vecs)]
    bs = [b_v[pl.ds(j * _L, _L)] for j in range(n_vecs)]
    inv_d = 1.0 / float(d)
    lane = lax.broadcasted_iota(jnp.int32, (_L,), 0)
    bfly_idx = [lax.bitwise_xor(lane, jnp.full((_L,), k, jnp.int32))
                for k in (1, 2, 4, 8)]

    def lane_sum(x):
      for idx in bfly_idx:
        x = x + _shuffle16(x, idx)
      return x

    def ln_one(r):
      vs = []
      for j in range(n_vecs):
        v = rows_v[r, pl.ds(j * _L, _L)] * scale + pe_v[r, pl.ds(j * _L, _L)]
        vs.append(v)
      acc = vs[0]
      acc2 = vs[0] * vs[0]
      for j in range(1, n_vecs):
        acc = acc + vs[j]
        acc2 = acc2 + vs[j] * vs[j]
      mean_v = lane_sum(acc) * inv_d
      ex2_v = lane_sum(acc2) * inv_d
      var_v = ex2_v - mean_v * mean_v
      rstd_v = _rsqrt16(var_v + _EPS)
      for j in range(n_vecs):
        t = (vs[j] - mean_v) * rstd_v
        rows_v[r, pl.ds(j * _L, _L)] = t * gs[j] + bs[j]

    def ln_rows(r, carry):
      ln_one(r * 2)
      ln_one(r * 2 + 1)
      return carry

    lax.fori_loop(0, rows_per_w // 2, ln_rows, 0)
    pltpu.sync_copy(rows_v, out_hbm.at[pl.ds(base, rows_per_w)])

  return body(ids2d, table, pe, gamma, beta)


def kernel(input_ids, token_table, pe, ln_gamma, ln_beta):
  b, s = input_ids.shape
  v, d = token_table.shape
  n_tokens = b * s
  info = plsc.get_sparse_core_info()
  n_workers = info.num_cores * info.num_subcores
  ids2d = input_ids.reshape(n_tokens // _IDX_CHUNK, _IDX_CHUNK).astype(
      jnp.int32)
  out = _embed_ln_call(n_tokens, s, d, n_workers, ids2d, token_table, pe,
                       ln_gamma, ln_beta)
  return out.reshape(b, s, d)


# speedup vs baseline: 1.1917x; 1.0106x over previous
import functools
import math

import jax
import jax.numpy as jnp
from jax import lax
from jax.experimental import pallas as pl
from jax.experimental.pallas import tpu as pltpu
from jax.experimental.pallas import tpu_sc as plsc

_EPS = 1e-5
_L = 16
_N_CHUNKS = 4

_GATHER_DNUMS = lax.GatherDimensionNumbers(
    offset_dims=(), collapsed_slice_dims=(0,), start_index_map=(0,))


def _shuffle16(x, idx):
  return lax.gather(
      x,
      idx[:, None],
      _GATHER_DNUMS,
      slice_sizes=(1,),
      mode=lax.GatherScatterMode.PROMISE_IN_BOUNDS)


def _rsqrt16(x):
  i = lax.bitcast_convert_type(x, jnp.int32)
  i = jnp.full((_L,), 0x5F3759DF, jnp.int32) - lax.shift_right_arithmetic(
      i, jnp.full((_L,), 1, jnp.int32))
  y = lax.bitcast_convert_type(i, jnp.float32)
  half_x = x * 0.5
  for _ in range(2):
    y = y * (1.5 - half_x * y * y)
  return y


@functools.partial(jax.jit, static_argnums=(0, 1))
def _embed_ln_call(d, n_workers, ids, table, pe, gamma, beta):
  b, s = ids.shape
  n_tokens = b * s
  rows_per_w = n_tokens // n_workers
  rows_per_c = rows_per_w // _N_CHUNKS
  n_vecs = d // _L
  scale = math.sqrt(float(d))
  mesh = plsc.VectorSubcoreMesh(core_axis_name="c", subcore_axis_name="s")

  @functools.partial(
      pl.kernel,
      mesh=mesh,
      out_type=jax.ShapeDtypeStruct((n_tokens, d), jnp.float32),
      scratch_types=[
          pltpu.VMEM((rows_per_w,), jnp.int32),
          pltpu.VMEM((rows_per_w, d), jnp.float32),
          pltpu.VMEM((rows_per_w, d), jnp.float32),
          pltpu.VMEM((d,), jnp.float32),
          pltpu.VMEM((d,), jnp.float32),
          pltpu.SemaphoreType.DMA,
          pltpu.SemaphoreType.DMA,
          pltpu.SemaphoreType.DMA,
          pltpu.SemaphoreType.DMA,
          pltpu.SemaphoreType.DMA,
          pltpu.SemaphoreType.DMA,
      ],
  )
  def body(ids_hbm, table_hbm, pe_hbm, gamma_hbm, beta_hbm, out_hbm, idx_v,
           rows_v, pe_v, g_v, b_v, gsem0, gsem1, gsem2, gsem3, pesem, wsem):
    gsems = [gsem0, gsem1, gsem2, gsem3]
    nc = lax.axis_size("c")
    wid = lax.axis_index("s") * nc + lax.axis_index("c")
    base = wid * rows_per_w
    bid = lax.div(base, s)
    pos0 = lax.rem(base, s)

    pltpu.sync_copy(ids_hbm.at[bid, pl.ds(pos0, rows_per_w)], idx_v)
    gcopies = []
    for c in range(_N_CHUNKS):
      gcopies.append(
          pltpu.async_copy(table_hbm.at[idx_v.at[pl.ds(c * rows_per_c,
                                                       rows_per_c)]],
                           rows_v.at[pl.ds(c * rows_per_c, rows_per_c)],
                           gsems[c]))
    pe_copy = pltpu.async_copy(pe_hbm.at[pl.ds(pos0, rows_per_w)], pe_v,
                               pesem)
    pltpu.sync_copy(gamma_hbm, g_v)
    pltpu.sync_copy(beta_hbm, b_v)
    pe_copy.wait()

    gs = [g_v[pl.ds(j * _L, _L)] for j in range(n_vecs)]
    bs = [b_v[pl.ds(j * _L, _L)] for j in range(n_vecs)]
    inv_d = 1.0 / float(d)
    lane = lax.broadcasted_iota(jnp.int32, (_L,), 0)
    bfly_idx = [lax.bitwise_xor(lane, jnp.full((_L,), k, jnp.int32))
                for k in (1, 2, 4, 8)]

    def lane_sum(x):
      for idx in bfly_idx:
        x = x + _shuffle16(x, idx)
      return x

    def ln_one(r):
      vs = []
      for j in range(n_vecs):
        v = rows_v[r, pl.ds(j * _L, _L)] * scale + pe_v[r, pl.ds(j * _L, _L)]
        vs.append(v)
      acc = vs[0]
      acc2 = vs[0] * vs[0]
      for j in range(1, n_vecs):
        acc = acc + vs[j]
        acc2 = acc2 + vs[j] * vs[j]
      mean_v = lane_sum(acc) * inv_d
      ex2_v = lane_sum(acc2) * inv_d
      var_v = ex2_v - mean_v * mean_v
      rstd_v = _rsqrt16(var_v + _EPS)
      for j in range(n_vecs):
        t = (vs[j] - mean_v) * rstd_v
        rows_v[r, pl.ds(j * _L, _L)] = t * gs[j] + bs[j]

    wcopies = []
    for c in range(_N_CHUNKS):
      gcopies[c].wait()
      r0 = c * rows_per_c

      @plsc.parallel_loop(0, rows_per_c, unroll=2)
      def _(r):
        ln_one(r0 + r)

      wcopies.append(
          pltpu.async_copy(rows_v.at[pl.ds(r0, rows_per_c)],
                           out_hbm.at[pl.ds(base + r0, rows_per_c)], wsem))
    for cp in wcopies:
      cp.wait()

  return body(ids, table, pe, gamma, beta)


def kernel(input_ids, token_table, pe, ln_gamma, ln_beta):
  b, s = input_ids.shape
  v, d = token_table.shape
  info = plsc.get_sparse_core_info()
  n_workers = info.num_cores * info.num_subcores
  out = _embed_ln_call(d, n_workers, input_ids.astype(jnp.int32), token_table,
                       pe, ln_gamma, ln_beta)
  return out.reshape(b, s, d)
